# baseline (speedup 1.0000x reference)
import functools

import jax
import jax.numpy as jnp
from jax import lax
from jax.experimental import pallas as pl
from jax.experimental.pallas import tpu as pltpu
from jax.experimental.pallas import tpu_sc as plsc

N = 10000
E = 320000
D_IN = 128
D_H = 32

NC = 2
NS = 16
NW = NC * NS

BLK = 128
BPW = 80
TOTB = NW * BPW
EPAD = TOTB * BLK
NROWS = 10112
RPT = NROWS // NS
TRASH = N
DEGW = 16

_f32 = jnp.float32
_mesh = plsc.VectorSubcoreMesh(core_axis_name="c", subcore_axis_name="s")
_sc_params = pltpu.CompilerParams(use_tc_tiling_on_sc=False)


def _deg_body(dstb, zdeg, ones_hbm, out, idx_d, ones_v, acc):
    c = lax.axis_index("c")
    s = lax.axis_index("s")
    w = c * NS + s
    r0 = s * RPT
    pltpu.sync_copy(zdeg.at[pl.ds(r0, RPT)], acc.at[pl.ds(r0, RPT)])
    pltpu.sync_copy(ones_hbm, ones_v)
    pltpu.sync_copy(dstb.at[pl.ds(w * BPW, BPW)], idx_d)
    plsc.subcore_barrier()

    def body(j, carry):
        pltpu.sync_copy(ones_v, acc.at[idx_d.at[j]], add=True)
        return carry

    lax.fori_loop(0, BPW, body, 0, unroll=False)
    plsc.subcore_barrier()
    pltpu.sync_copy(acc.at[pl.ds(r0, RPT)], out.at[c, pl.ds(r0, RPT)])


_deg_call = functools.partial(
    pl.kernel,
    out_type=jax.ShapeDtypeStruct((NC, NROWS, DEGW), _f32),
    mesh=_mesh,
    compiler_params=_sc_params,
    scratch_types=[
        pltpu.VMEM((BPW, BLK), jnp.int32),
        pltpu.VMEM((BLK, DEGW), _f32),
        pltpu.VMEM_SHARED((NROWS, DEGW), _f32),
    ],
)(_deg_body)


NBUF = 4


def _agg_body(g_hbm, srcb, dstb, zacc, out, idx_s, idx_d, rows, acc,
              g0, g1, g2, g3, s0, s1, s2, s3):
    gsem = (g0, g1, g2, g3)
    ssem = (s0, s1, s2, s3)
    c = lax.axis_index("c")
    s = lax.axis_index("s")
    w = c * NS + s
    r0 = s * RPT
    pltpu.sync_copy(zacc.at[pl.ds(r0, RPT)], acc.at[pl.ds(r0, RPT)])
    pltpu.sync_copy(srcb.at[pl.ds(w * BPW, BPW)], idx_s.at[pl.ds(0, BPW)])
    pltpu.sync_copy(dstb.at[pl.ds(w * BPW, BPW)], idx_d)
    z16 = jnp.zeros((16,), jnp.int32)
    for r in range(BPW, BPW + NBUF):
        for m in range(BLK // 16):
            idx_s[r, pl.ds(m * 16, 16)] = z16
    plsc.subcore_barrier()

    for b in range(NBUF):
        pltpu.async_copy(g_hbm.at[idx_s.at[b]], rows.at[b], gsem[b])

    def body(i, carry):
        j = i * NBUF
        for b in range(NBUF):
            pltpu.make_async_copy(
                g_hbm.at[idx_s.at[j + b]], rows.at[b], gsem[b]).wait()
            pltpu.async_copy(
                rows.at[b], acc.at[idx_d.at[j + b]], ssem[b], add=True)
        for b in range(NBUF):
            pltpu.make_async_copy(
                rows.at[b], acc.at[idx_d.at[j + b]], ssem[b]).wait()
            pltpu.async_copy(
                g_hbm.at[idx_s.at[j + NBUF + b]], rows.at[b], gsem[b])
        return carry

    lax.fori_loop(0, BPW // NBUF, body, 0, unroll=False)
    for b in range(NBUF):
        pltpu.make_async_copy(
            g_hbm.at[idx_s.at[BPW + b]], rows.at[b], gsem[b]).wait()
    plsc.subcore_barrier()
    pltpu.sync_copy(acc.at[pl.ds(r0, RPT)], out.at[c, pl.ds(r0, RPT)])


_agg_call = functools.partial(
    pl.kernel,
    out_type=jax.ShapeDtypeStruct((NC, NROWS, D_H), _f32),
    mesh=_mesh,
    compiler_params=_sc_params,
    scratch_types=[
        pltpu.VMEM((BPW + NBUF, BLK), jnp.int32),
        pltpu.VMEM((BPW, BLK), jnp.int32),
        pltpu.VMEM((NBUF, BLK, D_H), _f32),
        pltpu.VMEM_SHARED((NROWS, D_H), _f32),
    ] + [pltpu.SemaphoreType.DMA] * (2 * NBUF),
)(_agg_body)


def _mm_body(x_ref, w_ref, o_ref):
    o_ref[...] = jnp.dot(x_ref[...], w_ref[...], preferred_element_type=_f32)


def _mm_call(xp, W1):
    return pl.pallas_call(
        _mm_body,
        out_shape=jax.ShapeDtypeStruct((NROWS, D_H), _f32),
    )(xp, W1)


def _g_body(h_ref, degp_ref, g_ref, dinv_ref):
    deg = jnp.sum(degp_ref[0] + degp_ref[1], axis=1, keepdims=True) * (1.0 / DEGW) + 1.0
    dinv = lax.rsqrt(jnp.maximum(deg, 1.0))
    dinv_ref[...] = dinv
    g_ref[...] = h_ref[...] * dinv


def _g_call(h1, degp):
    return pl.pallas_call(
        _g_body,
        out_shape=(
            jax.ShapeDtypeStruct((NROWS, D_H), _f32),
            jax.ShapeDtypeStruct((NROWS, 1), _f32),
        ),
    )(h1, degp)


def _mid_body(p_ref, g_ref, dinv_ref, b_ref, w_ref, o_ref):
    out1 = (p_ref[0] + p_ref[1] + g_ref[...]) * dinv_ref[...] + b_ref[...]
    h2 = jnp.maximum(out1, 0.0)
    g2 = jnp.dot(h2, w_ref[...], preferred_element_type=_f32) * dinv_ref[...]
    row = lax.broadcasted_iota(jnp.int32, (NROWS, 1), 0)
    o_ref[...] = jnp.where(row < N, g2, 0.0)


def _mid_call(p1, g1, dinv, b1, W2):
    return pl.pallas_call(
        _mid_body,
        out_shape=jax.ShapeDtypeStruct((NROWS, D_H), _f32),
    )(p1, g1, dinv, b1.reshape(1, D_H), W2)


def _out_body(p_ref, g_ref, dinv_ref, b_ref, o_ref):
    o_ref[...] = (p_ref[0] + p_ref[1] + g_ref[...]) * dinv_ref[...] + b_ref[...]


def _out_call(p2, g2, dinv, b2):
    return pl.pallas_call(
        _out_body,
        out_shape=jax.ShapeDtypeStruct((NROWS, D_H), _f32),
    )(p2, g2, dinv, b2.reshape(1, D_H))


def kernel(x, edge_index, W1, b1, W2, b2):
    pad = EPAD - E
    srcp = jnp.concatenate(
        [edge_index[0], jnp.zeros((pad,), jnp.int32)]).reshape(TOTB, BLK)
    dstp = jnp.concatenate(
        [edge_index[1], jnp.full((pad,), TRASH, jnp.int32)]).reshape(TOTB, BLK)
    zacc = jnp.zeros((NROWS, D_H), _f32)
    zdeg = jnp.zeros((NROWS, DEGW), _f32)
    ones = jnp.ones((BLK, DEGW), _f32)
    xp = jnp.pad(x, ((0, NROWS - N), (0, 0)))

    degp = _deg_call(dstp, zdeg, ones)
    h1 = _mm_call(xp, W1)
    g1, dinv = _g_call(h1, degp)
    p1 = _agg_call(g1, srcp, dstp, zacc)
    g2 = _mid_call(p1, g1, dinv, b1, W2)
    p2 = _agg_call(g2, srcp, dstp, zacc)
    out = _out_call(p2, g2, dinv, b2)
    return out[:N]


# speedup vs baseline: 1.0142x; 1.0142x over previous
import functools

import jax
import jax.numpy as jnp
from jax import lax
from jax.experimental import pallas as pl
from jax.experimental.pallas import tpu as pltpu
from jax.experimental.pallas import tpu_sc as plsc

N = 10000
E = 320000
D_IN = 128
D_H = 32

NC = 2
NS = 16
NW = NC * NS

BLK = 128
BPW = 80
TOTB = NW * BPW
EPAD = TOTB * BLK
NROWS = 10112
RPT = NROWS // NS
TRASH = N
DEGW = 16

_f32 = jnp.float32
_mesh = plsc.VectorSubcoreMesh(core_axis_name="c", subcore_axis_name="s")
_sc_params = pltpu.CompilerParams(use_tc_tiling_on_sc=False)


def _deg_body(dstb, zdeg, ones_hbm, out, idx_d, ones_v, acc):
    c = lax.axis_index("c")
    s = lax.axis_index("s")
    w = c * NS + s
    r0 = s * RPT
    pltpu.sync_copy(zdeg.at[pl.ds(r0, RPT)], acc.at[pl.ds(r0, RPT)])
    pltpu.sync_copy(ones_hbm, ones_v)
    pltpu.sync_copy(dstb.at[pl.ds(w * BPW, BPW)], idx_d)
    plsc.subcore_barrier()

    def body(j, carry):
        pltpu.sync_copy(ones_v, acc.at[idx_d.at[j]], add=True)
        return carry

    lax.fori_loop(0, BPW, body, 0, unroll=False)
    plsc.subcore_barrier()
    pltpu.sync_copy(acc.at[pl.ds(r0, RPT)], out.at[c, pl.ds(r0, RPT)])


_deg_call = functools.partial(
    pl.kernel,
    out_type=jax.ShapeDtypeStruct((NC, NROWS, DEGW), _f32),
    mesh=_mesh,
    compiler_params=_sc_params,
    scratch_types=[
        pltpu.VMEM((BPW, BLK), jnp.int32),
        pltpu.VMEM((BLK, DEGW), _f32),
        pltpu.VMEM_SHARED((NROWS, DEGW), _f32),
    ],
)(_deg_body)


NBUF = 4


def _agg_body(g_hbm, srcb, dstb, zacc, out, idx_s, idx_d, rows, acc,
              g0, g1, g2, g3, s0, s1, s2, s3):
    gsem = (g0, g1, g2, g3)
    ssem = (s0, s1, s2, s3)
    c = lax.axis_index("c")
    s = lax.axis_index("s")
    w = c * NS + s
    r0 = s * RPT
    pltpu.sync_copy(zacc.at[pl.ds(r0, RPT)], acc.at[pl.ds(r0, RPT)])
    pltpu.sync_copy(srcb.at[pl.ds(w * BPW, BPW)], idx_s.at[pl.ds(0, BPW)])
    pltpu.sync_copy(dstb.at[pl.ds(w * BPW, BPW)], idx_d)
    z16 = jnp.zeros((16,), jnp.int32)
    for r in range(BPW, BPW + NBUF):
        for m in range(BLK // 16):
            idx_s[r, pl.ds(m * 16, 16)] = z16
    plsc.subcore_barrier()

    del ssem
    for b in range(NBUF):
        pltpu.async_copy(g_hbm.at[idx_s.at[b]], rows.at[b], gsem[b])

    def body(i, carry):
        j = i * NBUF
        for b in range(NBUF):
            pltpu.make_async_copy(
                g_hbm.at[idx_s.at[j + b]], rows.at[b], gsem[b]).wait()
            pltpu.sync_copy(rows.at[b], acc.at[idx_d.at[j + b]], add=True)
            pltpu.async_copy(
                g_hbm.at[idx_s.at[j + NBUF + b]], rows.at[b], gsem[b])
        return carry

    lax.fori_loop(0, BPW // NBUF, body, 0, unroll=False)
    for b in range(NBUF):
        pltpu.make_async_copy(
            g_hbm.at[idx_s.at[BPW + b]], rows.at[b], gsem[b]).wait()
    plsc.subcore_barrier()
    pltpu.sync_copy(acc.at[pl.ds(r0, RPT)], out.at[c, pl.ds(r0, RPT)])


_agg_call = functools.partial(
    pl.kernel,
    out_type=jax.ShapeDtypeStruct((NC, NROWS, D_H), _f32),
    mesh=_mesh,
    compiler_params=_sc_params,
    scratch_types=[
        pltpu.VMEM((BPW + NBUF, BLK), jnp.int32),
        pltpu.VMEM((BPW, BLK), jnp.int32),
        pltpu.VMEM((NBUF, BLK, D_H), _f32),
        pltpu.VMEM_SHARED((NROWS, D_H), _f32),
    ] + [pltpu.SemaphoreType.DMA] * (2 * NBUF),
)(_agg_body)


def _mm_body(x_ref, w_ref, o_ref):
    o_ref[...] = jnp.dot(x_ref[...], w_ref[...], preferred_element_type=_f32)


def _mm_call(xp, W1):
    return pl.pallas_call(
        _mm_body,
        out_shape=jax.ShapeDtypeStruct((NROWS, D_H), _f32),
    )(xp, W1)


def _g_body(h_ref, degp_ref, g_ref, dinv_ref):
    deg = jnp.sum(degp_ref[0] + degp_ref[1], axis=1, keepdims=True) * (1.0 / DEGW) + 1.0
    dinv = lax.rsqrt(jnp.maximum(deg, 1.0))
    dinv_ref[...] = dinv
    g_ref[...] = h_ref[...] * dinv


def _g_call(h1, degp):
    return pl.pallas_call(
        _g_body,
        out_shape=(
            jax.ShapeDtypeStruct((NROWS, D_H), _f32),
            jax.ShapeDtypeStruct((NROWS, 1), _f32),
        ),
    )(h1, degp)


def _mid_body(p_ref, g_ref, dinv_ref, b_ref, w_ref, o_ref):
    out1 = (p_ref[0] + p_ref[1] + g_ref[...]) * dinv_ref[...] + b_ref[...]
    h2 = jnp.maximum(out1, 0.0)
    g2 = jnp.dot(h2, w_ref[...], preferred_element_type=_f32) * dinv_ref[...]
    row = lax.broadcasted_iota(jnp.int32, (NROWS, 1), 0)
    o_ref[...] = jnp.where(row < N, g2, 0.0)


def _mid_call(p1, g1, dinv, b1, W2):
    return pl.pallas_call(
        _mid_body,
        out_shape=jax.ShapeDtypeStruct((NROWS, D_H), _f32),
    )(p1, g1, dinv, b1.reshape(1, D_H), W2)


def _out_body(p_ref, g_ref, dinv_ref, b_ref, o_ref):
    o_ref[...] = (p_ref[0] + p_ref[1] + g_ref[...]) * dinv_ref[...] + b_ref[...]


def _out_call(p2, g2, dinv, b2):
    return pl.pallas_call(
        _out_body,
        out_shape=jax.ShapeDtypeStruct((NROWS, D_H), _f32),
    )(p2, g2, dinv, b2.reshape(1, D_H))


def kernel(x, edge_index, W1, b1, W2, b2):
    pad = EPAD - E
    srcp = jnp.concatenate(
        [edge_index[0], jnp.zeros((pad,), jnp.int32)]).reshape(TOTB, BLK)
    dstp = jnp.concatenate(
        [edge_index[1], jnp.full((pad,), TRASH, jnp.int32)]).reshape(TOTB, BLK)
    zacc = jnp.zeros((NROWS, D_H), _f32)
    zdeg = jnp.zeros((NROWS, DEGW), _f32)
    ones = jnp.ones((BLK, DEGW), _f32)
    xp = jnp.pad(x, ((0, NROWS - N), (0, 0)))

    degp = _deg_call(dstp, zdeg, ones)
    h1 = _mm_call(xp, W1)
    g1, dinv = _g_call(h1, degp)
    p1 = _agg_call(g1, srcp, dstp, zacc)
    g2 = _mid_call(p1, g1, dinv, b1, W2)
    p2 = _agg_call(g2, srcp, dstp, zacc)
    out = _out_call(p2, g2, dinv, b2)
    return out[:N]


# speedup vs baseline: 1.7471x; 1.7227x over previous
import functools

import jax
import jax.numpy as jnp
from jax import lax
from jax.experimental import pallas as pl
from jax.experimental.pallas import tpu as pltpu
from jax.experimental.pallas import tpu_sc as plsc

N = 10000
E = 320000
D_IN = 128
D_H = 32

NC = 2
NS = 16
NW = NC * NS

BLK = 2560
BPW = 4
TOTB = NW * BPW
EPAD = TOTB * BLK
NROWS = 10112
RPT = NROWS // NS
TRASH = N
DEGW = 16

_f32 = jnp.float32
_mesh = plsc.VectorSubcoreMesh(core_axis_name="c", subcore_axis_name="s")
_sc_params = pltpu.CompilerParams(use_tc_tiling_on_sc=False)


def _deg_body(dstb, zdeg, ones_hbm, out, idx_d, ones_v, acc):
    c = lax.axis_index("c")
    s = lax.axis_index("s")
    w = c * NS + s
    r0 = s * RPT
    pltpu.sync_copy(zdeg.at[pl.ds(r0, RPT)], acc.at[pl.ds(r0, RPT)])
    pltpu.sync_copy(ones_hbm, ones_v)
    pltpu.sync_copy(dstb.at[pl.ds(w * BPW, BPW)], idx_d)
    plsc.subcore_barrier()

    def body(j, carry):
        pltpu.sync_copy(ones_v, acc.at[idx_d.at[j]], add=True)
        return carry

    lax.fori_loop(0, BPW, body, 0, unroll=False)
    plsc.subcore_barrier()
    pltpu.sync_copy(acc.at[pl.ds(r0, RPT)], out.at[c, pl.ds(r0, RPT)])


_deg_call = functools.partial(
    pl.kernel,
    out_type=jax.ShapeDtypeStruct((NC, NROWS, DEGW), _f32),
    mesh=_mesh,
    compiler_params=_sc_params,
    scratch_types=[
        pltpu.VMEM((BPW, BLK), jnp.int32),
        pltpu.VMEM((BLK, DEGW), _f32),
        pltpu.VMEM_SHARED((NROWS, DEGW), _f32),
    ],
)(_deg_body)


def _agg_body(g_hbm, srcb, dstb, zacc, out, idx_s, idx_d, rows, acc, sem):
    c = lax.axis_index("c")
    s = lax.axis_index("s")
    w = c * NS + s
    r0 = s * RPT
    pltpu.sync_copy(zacc.at[pl.ds(r0, RPT)], acc.at[pl.ds(r0, RPT)])
    pltpu.sync_copy(srcb.at[pl.ds(w * BPW, BPW)], idx_s)
    pltpu.sync_copy(dstb.at[pl.ds(w * BPW, BPW)], idx_d)
    plsc.subcore_barrier()

    def body(j, carry):
        pltpu.async_copy(g_hbm.at[idx_s.at[j]], rows, sem).wait()
        pltpu.sync_copy(rows, acc.at[idx_d.at[j]], add=True)
        return carry

    lax.fori_loop(0, BPW, body, 0, unroll=False)
    plsc.subcore_barrier()
    pltpu.sync_copy(acc.at[pl.ds(r0, RPT)], out.at[c, pl.ds(r0, RPT)])


_agg_call = functools.partial(
    pl.kernel,
    out_type=jax.ShapeDtypeStruct((NC, NROWS, D_H), _f32),
    mesh=_mesh,
    compiler_params=_sc_params,
    scratch_types=[
        pltpu.VMEM((BPW, BLK), jnp.int32),
        pltpu.VMEM((BPW, BLK), jnp.int32),
        pltpu.VMEM((BLK, D_H), _f32),
        pltpu.VMEM_SHARED((NROWS, D_H), _f32),
        pltpu.SemaphoreType.DMA,
    ],
)(_agg_body)


def _mm_body(x_ref, w_ref, o_ref):
    o_ref[...] = jnp.dot(x_ref[...], w_ref[...], preferred_element_type=_f32)


def _mm_call(xp, W1):
    return pl.pallas_call(
        _mm_body,
        out_shape=jax.ShapeDtypeStruct((NROWS, D_H), _f32),
    )(xp, W1)


def _g_body(h_ref, degp_ref, g_ref, dinv_ref):
    deg = jnp.sum(degp_ref[0] + degp_ref[1], axis=1, keepdims=True) * (1.0 / DEGW) + 1.0
    dinv = lax.rsqrt(jnp.maximum(deg, 1.0))
    dinv_ref[...] = dinv
    g_ref[...] = h_ref[...] * dinv


def _g_call(h1, degp):
    return pl.pallas_call(
        _g_body,
        out_shape=(
            jax.ShapeDtypeStruct((NROWS, D_H), _f32),
            jax.ShapeDtypeStruct((NROWS, 1), _f32),
        ),
    )(h1, degp)


def _mid_body(p_ref, g_ref, dinv_ref, b_ref, w_ref, o_ref):
    out1 = (p_ref[0] + p_ref[1] + g_ref[...]) * dinv_ref[...] + b_ref[...]
    h2 = jnp.maximum(out1, 0.0)
    g2 = jnp.dot(h2, w_ref[...], preferred_element_type=_f32) * dinv_ref[...]
    row = lax.broadcasted_iota(jnp.int32, (NROWS, 1), 0)
    o_ref[...] = jnp.where(row < N, g2, 0.0)


def _mid_call(p1, g1, dinv, b1, W2):
    return pl.pallas_call(
        _mid_body,
        out_shape=jax.ShapeDtypeStruct((NROWS, D_H), _f32),
    )(p1, g1, dinv, b1.reshape(1, D_H), W2)


def _out_body(p_ref, g_ref, dinv_ref, b_ref, o_ref):
    o_ref[...] = (p_ref[0] + p_ref[1] + g_ref[...]) * dinv_ref[...] + b_ref[...]


def _out_call(p2, g2, dinv, b2):
    return pl.pallas_call(
        _out_body,
        out_shape=jax.ShapeDtypeStruct((NROWS, D_H), _f32),
    )(p2, g2, dinv, b2.reshape(1, D_H))


def kernel(x, edge_index, W1, b1, W2, b2):
    pad = EPAD - E
    srcp = jnp.concatenate(
        [edge_index[0], jnp.zeros((pad,), jnp.int32)]).reshape(TOTB, BLK)
    dstp = jnp.concatenate(
        [edge_index[1], jnp.full((pad,), TRASH, jnp.int32)]).reshape(TOTB, BLK)
    zacc = jnp.zeros((NROWS, D_H), _f32)
    zdeg = jnp.zeros((NROWS, DEGW), _f32)
    ones = jnp.ones((BLK, DEGW), _f32)
    xp = jnp.pad(x, ((0, NROWS - N), (0, 0)))

    degp = _deg_call(dstp, zdeg, ones)
    h1 = _mm_call(xp, W1)
    g1, dinv = _g_call(h1, degp)
    p1 = _agg_call(g1, srcp, dstp, zacc)
    g2 = _mid_call(p1, g1, dinv, b1, W2)
    p2 = _agg_call(g2, srcp, dstp, zacc)
    out = _out_call(p2, g2, dinv, b2)
    return out[:N]
